# Initial kernel scaffold; baseline (speedup 1.0000x reference)
import functools

import jax
import jax.numpy as jnp
from jax import lax
from jax.experimental import pallas as pl
from jax.experimental.pallas import tpu as pltpu
from jax.experimental.pallas import tpu_sc as plsc

N = 10000
E = 320000
D = 128

NC = 2
NS = 16
NW = NC * NS
EPW = E // NW
CH = 125
NCH = EPW // CH
ROWS_PER_TILE = N // NS
ZR = 25


def _sc_agg_body(h_hbm, src_hbm, dst_hbm, out_hbm,
                 src_v, dst_v, rows_v, zbuf_v, acc_sh,
                 gsem0, gsem1, ssem0, ssem1):
    c = lax.axis_index("c")
    s = lax.axis_index("s")
    w = c * NS + s

    @pl.loop(0, ZR)
    def _zrow(i):
        for j in range(D // 16):
            zbuf_v[i, pl.ds(j * 16, 16)] = jnp.zeros((16,), jnp.float32)

    row_lo = s * ROWS_PER_TILE

    @pl.loop(0, ROWS_PER_TILE // ZR)
    def _zcopy(k):
        pltpu.sync_copy(zbuf_v, acc_sh.at[pl.ds(row_lo + k * ZR, ZR)])

    idx_base = w * NCH
    pltpu.sync_copy(src_hbm.at[pl.ds(idx_base, NCH)], src_v)
    pltpu.sync_copy(dst_hbm.at[pl.ds(idx_base, NCH)], dst_v)

    plsc.subcore_barrier()

    @pl.loop(0, NCH, step=2)
    def _chunk(i):
        g0 = pltpu.async_copy(h_hbm.at[src_v.at[i]], rows_v.at[0], gsem0)
        g1 = pltpu.async_copy(h_hbm.at[src_v.at[i + 1]], rows_v.at[1], gsem1)
        g0.wait()
        s0 = pltpu.async_copy(rows_v.at[0], acc_sh.at[dst_v.at[i]], ssem0,
                              add=True)
        g1.wait()
        s1 = pltpu.async_copy(rows_v.at[1], acc_sh.at[dst_v.at[i + 1]], ssem1,
                              add=True)
        s0.wait()
        s1.wait()

    plsc.subcore_barrier()

    pltpu.sync_copy(acc_sh.at[pl.ds(row_lo, ROWS_PER_TILE)],
                    out_hbm.at[pl.ds(c * N + row_lo, ROWS_PER_TILE)])


_sc_agg = functools.partial(
    pl.kernel,
    out_type=jax.ShapeDtypeStruct((2 * N, D), jnp.float32),
    mesh=plsc.VectorSubcoreMesh(core_axis_name="c", subcore_axis_name="s"),
    scratch_types=[
        pltpu.VMEM((NCH, CH), jnp.int32),
        pltpu.VMEM((NCH, CH), jnp.int32),
        pltpu.VMEM((2, CH, D), jnp.float32),
        pltpu.VMEM((ZR, D), jnp.float32),
        pltpu.VMEM_SHARED((N, D), jnp.float32),
        pltpu.SemaphoreType.DMA,
        pltpu.SemaphoreType.DMA,
        pltpu.SemaphoreType.DMA,
        pltpu.SemaphoreType.DMA,
    ],
)(_sc_agg_body)


def _tc_body(h_ref, p_ref, wa_ref, ba_ref, wb_ref, bb_ref, g_ref, be_ref,
             o_ref):
    x = h_ref[...] + p_ref[:N, :] + p_ref[N:, :]
    h1 = jnp.dot(x, wa_ref[...], preferred_element_type=jnp.float32)
    h1 = jnp.maximum(h1 + ba_ref[...], 0.0)
    h2 = jnp.dot(h1, wb_ref[...], preferred_element_type=jnp.float32)
    h2 = h2 + bb_ref[...]
    mu = jnp.mean(h2, axis=0, keepdims=True)
    d = h2 - mu
    var = jnp.mean(d * d, axis=0, keepdims=True)
    o_ref[...] = d * lax.rsqrt(var + 1e-5) * g_ref[...] + be_ref[...]


def _tc_stage(h, parts, Wa, ba, Wb, bb, g, be):
    return pl.pallas_call(
        _tc_body,
        out_shape=jax.ShapeDtypeStruct((N, D), jnp.float32),
    )(h, parts, Wa, ba.reshape(1, D), Wb, bb.reshape(1, D),
      g.reshape(1, D), be.reshape(1, D))


def kernel(x, edge_index, batch, W0a, b0a, W0b, b0b, g0, be0,
           W1a, b1a, W1b, b1b, g1, be1,
           W2a, b2a, W2b, b2b, g2, be2):
    src = edge_index[0].reshape(E // CH, CH)
    dst = edge_index[1].reshape(E // CH, CH)
    params = [(W0a, b0a, W0b, b0b, g0, be0),
              (W1a, b1a, W1b, b1b, g1, be1),
              (W2a, b2a, W2b, b2b, g2, be2)]
    h = x
    for (Wa, ba, Wb, bb, g, b) in params:
        parts = _sc_agg(h, src, dst)
        h = _tc_stage(h, parts, Wa, ba, Wb, bb, g, b)
    return h


# speedup vs baseline: 8.8266x; 8.8266x over previous
import functools

import jax
import jax.numpy as jnp
from jax import lax
from jax.experimental import pallas as pl
from jax.experimental.pallas import tpu as pltpu
from jax.experimental.pallas import tpu_sc as plsc

N = 10000
E = 320000
D = 128

NC = 2
NS = 16
NW = NC * NS
EPW = E // NW
CH = 125
NCH = EPW // CH
SEG = 40
ROWS_PER_TILE = 624
REM_LO = NS * ROWS_PER_TILE
REM_ROWS = N - REM_LO
ZR = 16


def _sc_agg_body(h_hbm, src_hbm, dst_hbm, out_hbm,
                 src_v, dst_v, rows_v, zbuf_v, acc_sh,
                 gsem0, gsem1, ssem0, ssem1):
    c = lax.axis_index("c")
    s = lax.axis_index("s")
    w = c * NS + s

    @pl.loop(0, ZR)
    def _zrow(i):
        for j in range(D // 16):
            zbuf_v[i, pl.ds(j * 16, 16)] = jnp.zeros((16,), jnp.float32)

    row_lo = s * ROWS_PER_TILE

    @pl.loop(0, ROWS_PER_TILE // ZR)
    def _zcopy(k):
        pltpu.sync_copy(zbuf_v, acc_sh.at[pl.ds(row_lo + k * ZR, ZR)])

    @pl.when(s == NS - 1)
    def _zrem():
        pltpu.sync_copy(zbuf_v, acc_sh.at[pl.ds(REM_LO, REM_ROWS)])

    idx_base = w * NCH

    plsc.subcore_barrier()

    @pl.loop(0, NCH // SEG)
    def _seg(g):
        pltpu.sync_copy(src_hbm.at[pl.ds(idx_base + g * SEG, SEG)], src_v)
        pltpu.sync_copy(dst_hbm.at[pl.ds(idx_base + g * SEG, SEG)], dst_v)

        @pl.loop(0, SEG, step=2)
        def _chunk(i):
            g0 = pltpu.async_copy(h_hbm.at[src_v.at[i]], rows_v.at[0], gsem0)
            g1 = pltpu.async_copy(h_hbm.at[src_v.at[i + 1]], rows_v.at[1],
                                  gsem1)
            g0.wait()
            s0 = pltpu.async_copy(rows_v.at[0], acc_sh.at[dst_v.at[i]], ssem0,
                                  add=True)
            g1.wait()
            s1 = pltpu.async_copy(rows_v.at[1], acc_sh.at[dst_v.at[i + 1]],
                                  ssem1, add=True)
            s0.wait()
            s1.wait()

    plsc.subcore_barrier()

    pltpu.sync_copy(acc_sh.at[pl.ds(row_lo, ROWS_PER_TILE)],
                    out_hbm.at[pl.ds(c * N + row_lo, ROWS_PER_TILE)])

    @pl.when(s == NS - 1)
    def _orem():
        pltpu.sync_copy(acc_sh.at[pl.ds(REM_LO, REM_ROWS)],
                        out_hbm.at[pl.ds(c * N + REM_LO, REM_ROWS)])


_sc_agg = functools.partial(
    pl.kernel,
    out_type=jax.ShapeDtypeStruct((2 * N, D), jnp.float32),
    mesh=plsc.VectorSubcoreMesh(core_axis_name="c", subcore_axis_name="s"),
    scratch_types=[
        pltpu.VMEM((SEG, CH), jnp.int32),
        pltpu.VMEM((SEG, CH), jnp.int32),
        pltpu.VMEM((2, CH, D), jnp.float32),
        pltpu.VMEM((ZR, D), jnp.float32),
        pltpu.VMEM_SHARED((N, D), jnp.float32),
        pltpu.SemaphoreType.DMA,
        pltpu.SemaphoreType.DMA,
        pltpu.SemaphoreType.DMA,
        pltpu.SemaphoreType.DMA,
    ],
)(_sc_agg_body)


def _tc_body(h_ref, p_ref, wa_ref, ba_ref, wb_ref, bb_ref, g_ref, be_ref,
             o_ref):
    x = h_ref[...] + p_ref[:N, :] + p_ref[N:, :]
    h1 = jnp.dot(x, wa_ref[...], preferred_element_type=jnp.float32)
    h1 = jnp.maximum(h1 + ba_ref[...], 0.0)
    h2 = jnp.dot(h1, wb_ref[...], preferred_element_type=jnp.float32)
    h2 = h2 + bb_ref[...]
    mu = jnp.mean(h2, axis=0, keepdims=True)
    d = h2 - mu
    var = jnp.mean(d * d, axis=0, keepdims=True)
    o_ref[...] = d * lax.rsqrt(var + 1e-5) * g_ref[...] + be_ref[...]


def _tc_stage(h, parts, Wa, ba, Wb, bb, g, be):
    return pl.pallas_call(
        _tc_body,
        out_shape=jax.ShapeDtypeStruct((N, D), jnp.float32),
    )(h, parts, Wa, ba.reshape(1, D), Wb, bb.reshape(1, D),
      g.reshape(1, D), be.reshape(1, D))


def kernel(x, edge_index, batch, W0a, b0a, W0b, b0b, g0, be0,
           W1a, b1a, W1b, b1b, g1, be1,
           W2a, b2a, W2b, b2b, g2, be2):
    src = edge_index[0].reshape(E // CH, CH)
    dst = edge_index[1].reshape(E // CH, CH)
    params = [(W0a, b0a, W0b, b0b, g0, be0),
              (W1a, b1a, W1b, b1b, g1, be1),
              (W2a, b2a, W2b, b2b, g2, be2)]
    h = x
    for (Wa, ba, Wb, bb, g, b) in params:
        parts = _sc_agg(h, src, dst)
        h = _tc_stage(h, parts, Wa, ba, Wb, bb, g, b)
    return h
